# baseline (speedup 1.0000x reference)
import jax
import jax.numpy as jnp
from jax.experimental import pallas as pl
from jax.experimental.pallas import tpu as pltpu

_STRENGTH_DECAY = 0.995
_AGE_PENALTY = 0.02
_STRENGTH_BOOST = 0.5
_WRITE_ALPHA = 0.5
_WRITE_TEMP = 50.0
_EVICT_AGE_BOOST = 0.05
_EVICT_STR_PENALTY = 0.5


def _round_bf16(x):
    i = jax.lax.bitcast_convert_type(x, jnp.uint32)
    r = (i + jnp.uint32(0x7FFF) + ((i >> 16) & jnp.uint32(1))) & jnp.uint32(
        0xFFFF0000)
    return jax.lax.bitcast_convert_type(r, jnp.float32)


def _block_body(q_ref, w_ref, ws_ref, keys_ref, vals_ref, age_ref, str_ref,
                read_ref, kout_ref, vout_ref, ageout_ref, strout_ref):
    q = q_ref[...]
    wv = w_ref[...]
    ws = ws_ref[...]
    keys = keys_ref[...]
    vals = vals_ref[...]
    age = age_ref[...]
    stg = str_ref[...]
    K = keys.shape[1]

    q_den = jnp.sqrt(jnp.sum(q * q, axis=-1, keepdims=True)) + 1e-6
    w_den = jnp.sqrt(jnp.sum(wv * wv, axis=-1, keepdims=True)) + 1e-6
    k_den = jnp.sqrt(jnp.sum(keys * keys, axis=-1, keepdims=True)) + 1e-6
    qn = q / q_den
    wk_n = wv / w_den
    kn_b = _round_bf16(keys / k_den)
    qn_b = _round_bf16(qn)
    wn_b = _round_bf16(wk_n)

    sim_r = jnp.sum(kn_b * qn_b[:, None, :], axis=-1)
    sim_w = jnp.sum(kn_b * wn_b[:, None, :], axis=-1)

    logits = (sim_r + _STRENGTH_BOOST * jnp.log(jnp.clip(stg, 0.001, 1.0))
              - _AGE_PENALTY * age)
    m_r = jnp.max(logits, axis=-1, keepdims=True)
    e_r = jnp.exp(logits - m_r)
    w_read = e_r / jnp.sum(e_r, axis=-1, keepdims=True)
    read_ref[...] = jnp.sum(
        _round_bf16(w_read)[:, :, None] * _round_bf16(vals), axis=1)

    wl = (sim_w * _WRITE_TEMP + _EVICT_AGE_BOOST * jnp.log1p(age)
          - _EVICT_STR_PENALTY * stg)
    m_w = jnp.max(wl, axis=-1, keepdims=True)
    e_w = jnp.exp(wl - m_w)
    soft = e_w / jnp.sum(e_w, axis=-1, keepdims=True)

    iota = jax.lax.broadcasted_iota(jnp.int32, soft.shape, 1)
    s_max = jnp.max(soft, axis=-1, keepdims=True)
    top = jnp.min(jnp.where(soft == s_max, iota, K), axis=-1, keepdims=True)
    hard = (iota == top).astype(jnp.float32)
    write_w = (hard - soft) + soft

    wsc = jnp.clip(ws, 0.0, 1.0)
    eff = write_w * wsc * _WRITE_ALPHA
    rate = eff[:, :, None]
    kout_ref[...] = (1.0 - rate) * keys + rate * wk_n[:, None, :]
    vout_ref[...] = (1.0 - rate) * vals + rate * wv[:, None, :]
    ageout_ref[...] = (age + 1.0) * (1.0 - write_w)
    s_new = stg * _STRENGTH_DECAY + write_w * wsc * (1.0 - stg * _STRENGTH_DECAY)
    strout_ref[...] = jnp.clip(s_new, 0.0, 1.0)


def kernel(query_vec, write_vec, write_strength, keys, vals, age, strength,
           *, interpret=False):
    B, D = query_vec.shape
    K = keys.shape[1]
    bB = 128 if B % 128 == 0 else 8
    grid = (B // bB,)

    row2 = lambda i: (i, 0)
    row3 = lambda i: (i, 0, 0)
    f32 = jnp.float32

    out_shapes = (
        jax.ShapeDtypeStruct((B, D), f32),
        jax.ShapeDtypeStruct((B, K, D), f32),
        jax.ShapeDtypeStruct((B, K, D), f32),
        jax.ShapeDtypeStruct((B, K), f32),
        jax.ShapeDtypeStruct((B, K), f32),
    )
    out_specs = (
        pl.BlockSpec((bB, D), row2),
        pl.BlockSpec((bB, K, D), row3),
        pl.BlockSpec((bB, K, D), row3),
        pl.BlockSpec((bB, K), row2),
        pl.BlockSpec((bB, K), row2),
    )
    in_specs = [
        pl.BlockSpec((bB, D), row2),
        pl.BlockSpec((bB, D), row2),
        pl.BlockSpec((bB, 1), row2),
        pl.BlockSpec((bB, K, D), row3),
        pl.BlockSpec((bB, K, D), row3),
        pl.BlockSpec((bB, K), row2),
        pl.BlockSpec((bB, K), row2),
    ]

    read_out, keys_new, vals_new, age_new, str_new = pl.pallas_call(
        _block_body,
        out_shape=out_shapes,
        grid=grid,
        in_specs=in_specs,
        out_specs=out_specs,
        compiler_params=pltpu.CompilerParams(
            dimension_semantics=("parallel",),
            vmem_limit_bytes=56 * 1024 * 1024,
        ),
        name="episodic_slot_memory",
        interpret=interpret,
    )(query_vec, write_vec, write_strength, keys, vals, age, strength)

    return (read_out, (keys_new, vals_new, age_new, str_new))


# speedup vs baseline: 1.1159x; 1.1159x over previous
import jax
import jax.numpy as jnp
from jax.experimental import pallas as pl
from jax.experimental.pallas import tpu as pltpu

_STRENGTH_DECAY = 0.995
_AGE_PENALTY = 0.02
_STRENGTH_BOOST = 0.5
_WRITE_ALPHA = 0.5
_WRITE_TEMP = 50.0
_EVICT_AGE_BOOST = 0.05
_EVICT_STR_PENALTY = 0.5


def _round_bf16(x):
    return x.astype(jnp.bfloat16).astype(jnp.float32)


def _block_body(q_ref, w_ref, ws_ref, keys_ref, vals_ref, age_ref, str_ref,
                read_ref, kout_ref, vout_ref, ageout_ref, strout_ref):
    q = q_ref[...]
    wv = w_ref[...]
    ws = ws_ref[...]
    keys = keys_ref[...]
    vals = vals_ref[...]
    age = age_ref[...]
    stg = str_ref[...]
    K = keys.shape[1]

    q_den = jnp.sqrt(jnp.sum(q * q, axis=-1, keepdims=True)) + 1e-6
    w_den = jnp.sqrt(jnp.sum(wv * wv, axis=-1, keepdims=True)) + 1e-6
    k_den = jnp.sqrt(jnp.sum(keys * keys, axis=-1, keepdims=True)) + 1e-6
    qn = q / q_den
    wk_n = wv / w_den
    kn_b = _round_bf16(keys / k_den)
    qn_b = _round_bf16(qn)
    wn_b = _round_bf16(wk_n)

    sim_r = jnp.sum(kn_b * qn_b[:, None, :], axis=-1)
    sim_w = jnp.sum(kn_b * wn_b[:, None, :], axis=-1)

    logits = (sim_r + _STRENGTH_BOOST * jnp.log(jnp.clip(stg, 0.001, 1.0))
              - _AGE_PENALTY * age)
    m_r = jnp.max(logits, axis=-1, keepdims=True)
    e_r = jnp.exp(logits - m_r)
    w_read = e_r / jnp.sum(e_r, axis=-1, keepdims=True)
    read_ref[...] = jnp.sum(
        _round_bf16(w_read)[:, :, None] * _round_bf16(vals), axis=1)

    wl = (sim_w * _WRITE_TEMP + _EVICT_AGE_BOOST * jnp.log1p(age)
          - _EVICT_STR_PENALTY * stg)
    m_w = jnp.max(wl, axis=-1, keepdims=True)
    e_w = jnp.exp(wl - m_w)
    soft = e_w / jnp.sum(e_w, axis=-1, keepdims=True)

    iota = jax.lax.broadcasted_iota(jnp.int32, soft.shape, 1)
    s_max = jnp.max(soft, axis=-1, keepdims=True)
    top = jnp.min(jnp.where(soft == s_max, iota, K), axis=-1, keepdims=True)
    hard = (iota == top).astype(jnp.float32)
    write_w = (hard - soft) + soft

    wsc = jnp.clip(ws, 0.0, 1.0)
    eff = write_w * wsc * _WRITE_ALPHA
    rate = eff[:, :, None]
    kout_ref[...] = (1.0 - rate) * keys + rate * wk_n[:, None, :]
    vout_ref[...] = (1.0 - rate) * vals + rate * wv[:, None, :]
    ageout_ref[...] = (age + 1.0) * (1.0 - write_w)
    s_new = stg * _STRENGTH_DECAY + write_w * wsc * (1.0 - stg * _STRENGTH_DECAY)
    strout_ref[...] = jnp.clip(s_new, 0.0, 1.0)


def kernel(query_vec, write_vec, write_strength, keys, vals, age, strength,
           *, interpret=False):
    B, D = query_vec.shape
    K = keys.shape[1]
    bB = 128 if B % 128 == 0 else 8
    grid = (B // bB,)

    row2 = lambda i: (i, 0)
    row3 = lambda i: (i, 0, 0)
    f32 = jnp.float32

    out_shapes = (
        jax.ShapeDtypeStruct((B, D), f32),
        jax.ShapeDtypeStruct((B, K, D), f32),
        jax.ShapeDtypeStruct((B, K, D), f32),
        jax.ShapeDtypeStruct((B, K), f32),
        jax.ShapeDtypeStruct((B, K), f32),
    )
    out_specs = (
        pl.BlockSpec((bB, D), row2),
        pl.BlockSpec((bB, K, D), row3),
        pl.BlockSpec((bB, K, D), row3),
        pl.BlockSpec((bB, K), row2),
        pl.BlockSpec((bB, K), row2),
    )
    in_specs = [
        pl.BlockSpec((bB, D), row2),
        pl.BlockSpec((bB, D), row2),
        pl.BlockSpec((bB, 1), row2),
        pl.BlockSpec((bB, K, D), row3),
        pl.BlockSpec((bB, K, D), row3),
        pl.BlockSpec((bB, K), row2),
        pl.BlockSpec((bB, K), row2),
    ]

    read_out, keys_new, vals_new, age_new, str_new = pl.pallas_call(
        _block_body,
        out_shape=out_shapes,
        grid=grid,
        in_specs=in_specs,
        out_specs=out_specs,
        compiler_params=pltpu.CompilerParams(
            dimension_semantics=("parallel",),
            vmem_limit_bytes=56 * 1024 * 1024,
        ),
        name="episodic_slot_memory",
        interpret=interpret,
    )(query_vec, write_vec, write_strength, keys, vals, age, strength)

    return (read_out, (keys_new, vals_new, age_new, str_new))
